# baseline (speedup 1.0000x reference)
import jax
import jax.numpy as jnp
import numpy as np
from jax import lax
from jax.experimental import pallas as pl
from jax.experimental.pallas import tpu as pltpu
from jax.experimental.pallas import tpu_sc as plsc

_N_CAT = 26
_N_NUM = 13
_NF = _N_NUM + _N_CAT
_D = 64
_BATCH = 16384
_CARD = 100000

_NC, _NS = 2, 16
_NW = _NC * _NS
_RPT = _BATCH // _NW
_L = 16


def _sc_body(xcT_h, table_h, out_h, idx_v, rowsA_v, rowsB_v, gsemA, gsemB):
    wid = lax.axis_index("s") * _NC + lax.axis_index("c")
    b0 = wid * _RPT

    pltpu.sync_copy(xcT_h.at[:, pl.ds(b0, _RPT)], idx_v)

    def add_off(v, c):
        for j in range(_N_CAT):
            idx_v[j, pl.ds(v * _L, _L)] = idx_v[j, pl.ds(v * _L, _L)] + j * _CARD
        return c
    lax.fori_loop(0, _RPT // _L, add_off, 0)

    def issue(j, rows_v, sem):
        for g in range(4):
            pltpu.async_copy(
                table_h.at[idx_v.at[j, pl.ds(g * 128, 128)]],
                rows_v.at[pl.ds(g * 128, 128), :],
                sem)

    def drain(rows_v, sem):
        pltpu.make_async_copy(table_h.at[pl.ds(0, _RPT), :], rows_v, sem).wait()

    issue(0, rowsA_v, gsemA)

    def pair(i, c):
        j = i * 2
        issue(j + 1, rowsB_v, gsemB)
        drain(rowsA_v, gsemA)
        pltpu.sync_copy(rowsA_v, out_h.at[pl.ds(j * _BATCH + b0, _RPT), :])

        @pl.when(j + 2 < _N_CAT)
        def _():
            issue(j + 2, rowsA_v, gsemA)
        drain(rowsB_v, gsemB)
        pltpu.sync_copy(rowsB_v, out_h.at[pl.ds((j + 1) * _BATCH + b0, _RPT), :])
        return c
    lax.fori_loop(0, _N_CAT // 2, pair, 0)


_BB = 16


def _tc_body(g_ref, xn_ref, w_ref, bn_ref, bias_ref, o_ref):
    f = pl.program_id(0)

    @pl.when(f < _N_NUM)
    def _():
        x2 = xn_ref[0, 0, :].reshape(_BB, 128)
        w = w_ref[0, 0, :]
        b = bn_ref[0, 0, :]
        vals = [(w[:, None] * x2[bb][None, :] + b[:, None]).reshape(8, 8, 128)
                for bb in range(_BB)]
        o_ref[...] = jnp.stack(vals, axis=1)[None]

    @pl.when(f >= _N_NUM)
    def _():
        ga = g_ref[...]
        bias = bias_ref[0, 0, :]
        eye = jnp.eye(_D, dtype=jnp.float32)
        vals = []
        for bb in range(_BB):
            m = ga[0, bb * 8:(bb + 1) * 8].reshape(_D, 128)
            t = jnp.concatenate(
                [lax.dot_general(m[:, :64], eye, (((0,), (0,)), ((), ())),
                                 preferred_element_type=jnp.float32),
                 lax.dot_general(m[:, 64:], eye, (((0,), (0,)), ((), ())),
                                 preferred_element_type=jnp.float32)],
                axis=1)
            vals.append((t + bias[:, None]).reshape(8, 8, 128))
        o_ref[...] = jnp.stack(vals, axis=1)[None]


def kernel(x_num, x_cat, W_num, b_num, table, cat_bias):
    mesh = plsc.VectorSubcoreMesh(core_axis_name="c", subcore_axis_name="s")
    sc_run = pl.kernel(
        _sc_body,
        out_type=jax.ShapeDtypeStruct((_N_CAT * _BATCH, _D), jnp.float32),
        mesh=mesh,
        compiler_params=pltpu.CompilerParams(use_tc_tiling_on_sc=False),
        scratch_types=[
            pltpu.VMEM((_N_CAT, _RPT), jnp.int32),
            pltpu.VMEM((_RPT, _D), jnp.float32),
            pltpu.VMEM((_RPT, _D), jnp.float32),
            pltpu.SemaphoreType.DMA,
            pltpu.SemaphoreType.DMA,
        ],
    )
    perm = jnp.asarray(np.arange(128) // 2 + 64 * (np.arange(128) % 2),
                       dtype=jnp.int32)
    xcp = (x_cat.T.reshape(_N_CAT, _BATCH // 128, 128)[:, :, perm]
           .reshape(_N_CAT, _BATCH))
    g = sc_run(xcp, table)
    g8 = g.reshape(_N_CAT, _BATCH * _D // 1024, 8, 128)

    nbb = _BATCH // 128
    out5 = pl.pallas_call(
        _tc_body,
        grid=(_NF, nbb // _BB),
        in_specs=[
            pl.BlockSpec((1, 8 * _BB, 8, 128),
                         lambda f, b: (jnp.maximum(f - _N_NUM, 0), b, 0, 0)),
            pl.BlockSpec((1, 1, 128 * _BB),
                         lambda f, b: (jnp.minimum(f, _N_NUM - 1), 0, b)),
            pl.BlockSpec((1, 1, _D),
                         lambda f, b: (jnp.minimum(f, _N_NUM - 1), 0, 0)),
            pl.BlockSpec((1, 1, _D),
                         lambda f, b: (jnp.minimum(f, _N_NUM - 1), 0, 0)),
            pl.BlockSpec((1, 1, _D),
                         lambda f, b: (jnp.maximum(f - _N_NUM, 0), 0, 0)),
        ],
        out_specs=pl.BlockSpec((1, 8, _BB, 8, 128),
                               lambda f, b: (f, 0, b, 0, 0)),
        out_shape=jax.ShapeDtypeStruct((_NF, 8, nbb, 8, 128), jnp.float32),
    )(g8,
      x_num.T.reshape(_N_NUM, 1, _BATCH),
      W_num.reshape(_N_NUM, 1, _D),
      b_num.reshape(_N_NUM, 1, _D),
      cat_bias.reshape(_N_CAT, 1, _D))

    return (out5.reshape(_NF, 8, 128, 8, 128)
                .transpose(2, 4, 0, 1, 3)
                .reshape(_BATCH, _NF, _D))


# speedup vs baseline: 1.2258x; 1.2258x over previous
import jax
import jax.numpy as jnp
import numpy as np
from jax import lax
from jax.experimental import pallas as pl
from jax.experimental.pallas import tpu as pltpu
from jax.experimental.pallas import tpu_sc as plsc

_N_CAT = 26
_N_NUM = 13
_NF = _N_NUM + _N_CAT
_D = 64
_BATCH = 16384
_CARD = 100000

_NC, _NS = 2, 16
_NW = _NC * _NS
_RPT = _BATCH // _NW
_L = 16


def _sc_body(xcT_h, table_h, out_h, idx_v, rowsA_v, rowsB_v, gsemA, gsemB):
    wid = lax.axis_index("s") * _NC + lax.axis_index("c")
    b0 = wid * _RPT

    pltpu.sync_copy(xcT_h.at[:, pl.ds(b0, _RPT)], idx_v)

    def add_off(v, c):
        for j in range(_N_CAT):
            idx_v[j, pl.ds(v * _L, _L)] = idx_v[j, pl.ds(v * _L, _L)] + j * _CARD
        return c
    lax.fori_loop(0, _RPT // _L, add_off, 0)

    def issue(j, rows_v, sem):
        for g in range(4):
            pltpu.async_copy(
                table_h.at[idx_v.at[j, pl.ds(g * 128, 128)]],
                rows_v.at[pl.ds(g * 128, 128), :],
                sem)

    def drain(rows_v, sem):
        pltpu.make_async_copy(table_h.at[pl.ds(0, _RPT), :], rows_v, sem).wait()

    issue(0, rowsA_v, gsemA)

    def pair(i, c):
        j = i * 2
        issue(j + 1, rowsB_v, gsemB)
        drain(rowsA_v, gsemA)
        pltpu.sync_copy(rowsA_v, out_h.at[pl.ds(j * _BATCH + b0, _RPT), :])

        @pl.when(j + 2 < _N_CAT)
        def _():
            issue(j + 2, rowsA_v, gsemA)
        drain(rowsB_v, gsemB)
        pltpu.sync_copy(rowsB_v, out_h.at[pl.ds((j + 1) * _BATCH + b0, _RPT), :])
        return c
    lax.fori_loop(0, _N_CAT // 2, pair, 0)


_BB = 16


def _tc_body(g_ref, xn_ref, w_ref, bn_ref, bias_ref, o_ref):
    f = pl.program_id(0)

    @pl.when(f < _N_NUM)
    def _():
        x2 = xn_ref[0, 0, :].reshape(_BB, 128)
        w = w_ref[0, 0, :]
        b = bn_ref[0, 0, :]
        vals = [(w[:, None] * x2[bb][None, :] + b[:, None]).reshape(8, 8, 128)
                for bb in range(_BB)]
        o_ref[...] = jnp.stack(vals, axis=1)[None]

    @pl.when(f >= _N_NUM)
    def _():
        ga = g_ref[...]
        bias = bias_ref[0, 0, :]
        vals = []
        for p in range(_BB // 2):
            mt = ga[0, p * 16:(p + 1) * 16].reshape(128, 128).T
            t0 = jnp.concatenate([mt[:64, :64], mt[64:, :64]], axis=1)
            t1 = jnp.concatenate([mt[:64, 64:], mt[64:, 64:]], axis=1)
            vals.append((t0 + bias[:, None]).reshape(8, 8, 128))
            vals.append((t1 + bias[:, None]).reshape(8, 8, 128))
        o_ref[...] = jnp.stack(vals, axis=1)[None]


def kernel(x_num, x_cat, W_num, b_num, table, cat_bias):
    mesh = plsc.VectorSubcoreMesh(core_axis_name="c", subcore_axis_name="s")
    sc_run = pl.kernel(
        _sc_body,
        out_type=jax.ShapeDtypeStruct((_N_CAT * _BATCH, _D), jnp.float32),
        mesh=mesh,
        compiler_params=pltpu.CompilerParams(use_tc_tiling_on_sc=False),
        scratch_types=[
            pltpu.VMEM((_N_CAT, _RPT), jnp.int32),
            pltpu.VMEM((_RPT, _D), jnp.float32),
            pltpu.VMEM((_RPT, _D), jnp.float32),
            pltpu.SemaphoreType.DMA,
            pltpu.SemaphoreType.DMA,
        ],
    )
    perm = jnp.asarray(np.arange(128) // 2 + 64 * (np.arange(128) % 2),
                       dtype=jnp.int32)
    xcp = (x_cat.T.reshape(_N_CAT, _BATCH // 128, 128)[:, :, perm]
           .reshape(_N_CAT, _BATCH))
    g = sc_run(xcp, table)
    g8 = g.reshape(_N_CAT, _BATCH * _D // 1024, 8, 128)

    nbb = _BATCH // 128
    out5 = pl.pallas_call(
        _tc_body,
        grid=(_NF, nbb // _BB),
        in_specs=[
            pl.BlockSpec((1, 8 * _BB, 8, 128),
                         lambda f, b: (jnp.maximum(f - _N_NUM, 0), b, 0, 0)),
            pl.BlockSpec((1, 1, 128 * _BB),
                         lambda f, b: (jnp.minimum(f, _N_NUM - 1), 0, b)),
            pl.BlockSpec((1, 1, _D),
                         lambda f, b: (jnp.minimum(f, _N_NUM - 1), 0, 0)),
            pl.BlockSpec((1, 1, _D),
                         lambda f, b: (jnp.minimum(f, _N_NUM - 1), 0, 0)),
            pl.BlockSpec((1, 1, _D),
                         lambda f, b: (jnp.maximum(f - _N_NUM, 0), 0, 0)),
        ],
        out_specs=pl.BlockSpec((1, 8, _BB, 8, 128),
                               lambda f, b: (f, 0, b, 0, 0)),
        out_shape=jax.ShapeDtypeStruct((_NF, 8, nbb, 8, 128), jnp.float32),
    )(g8,
      x_num.T.reshape(_N_NUM, 1, _BATCH),
      W_num.reshape(_N_NUM, 1, _D),
      b_num.reshape(_N_NUM, 1, _D),
      cat_bias.reshape(_N_CAT, 1, _D))

    return (out5.reshape(_NF, 8, 128, 8, 128)
                .transpose(2, 4, 0, 1, 3)
                .reshape(_BATCH, _NF, _D))
